# baseline (speedup 1.0000x reference)
import functools

import jax
import jax.numpy as jnp
from jax import lax
from jax.experimental import pallas as pl
from jax.experimental.pallas import tpu as pltpu
from jax.experimental.pallas import tpu_sc as plsc

_NC = 2
_NS = 16


@functools.lru_cache(maxsize=None)
def _make_sc_hop(N, D, E, CH=80):
    EPC = E // _NC
    EPT = EPC // _NS
    NCH = EPT // CH
    assert CH % 8 == 0 and NCH * CH == EPT and EPC * _NC == E
    RPT = (N // _NS) // 8 * 8
    TAIL = N - _NS * RPT
    assert TAIL % 8 == 0 and 0 <= TAIL <= 128

    mesh = plsc.VectorSubcoreMesh(
        core_axis_name="c", subcore_axis_name="s",
        num_cores=_NC, num_subcores=_NS)

    NBUF = 4
    assert NCH >= 6

    @functools.partial(
        pl.kernel,
        out_type=jax.ShapeDtypeStruct((_NC, N, D), jnp.float32),
        mesh=mesh,
        scratch_types=[
            [pltpu.VMEM((CH,), jnp.int32) for _ in range(NBUF)],
            [pltpu.VMEM((CH,), jnp.int32) for _ in range(NBUF)],
            [pltpu.VMEM((CH, D), jnp.float32) for _ in range(NBUF)],
            pltpu.VMEM((32, D), jnp.float32),
            pltpu.VMEM_SHARED((N, D), jnp.float32),
            [pltpu.SemaphoreType.DMA for _ in range(NBUF)],
            [pltpu.SemaphoreType.DMA for _ in range(NBUF)],
            [pltpu.SemaphoreType.DMA for _ in range(NBUF)],
        ],
    )
    def hop(p_hbm, src_hbm, dst_hbm, z_hbm, out_hbm,
            idx_s, idx_d, rows, zbuf, acc, gsem, ssem, isem):
        c = lax.axis_index("c")
        s = lax.axis_index("s")

        base = (c * _NS + s) * EPT

        def istart(i, b):
            e0 = base + i * CH
            pltpu.async_copy(src_hbm.at[pl.ds(e0, CH)], idx_s[b], isem[b])
            pltpu.async_copy(dst_hbm.at[pl.ds(e0, CH)], idx_d[b], isem[b])

        def iwait(i, b):
            e0 = base + i * CH
            pltpu.make_async_copy(
                src_hbm.at[pl.ds(e0, CH)], idx_s[b], isem[b]).wait()
            pltpu.make_async_copy(
                dst_hbm.at[pl.ds(e0, CH)], idx_d[b], isem[b]).wait()

        def gstart(b):
            pltpu.async_copy(p_hbm.at[idx_s[b]], rows[b], gsem[b])

        def gwait(b):
            pltpu.make_async_copy(p_hbm.at[idx_s[b]], rows[b], gsem[b]).wait()

        def sstart(b):
            pltpu.async_copy(rows[b], acc.at[idx_d[b]], ssem[b], add=True)

        def swait(b):
            pltpu.make_async_copy(rows[b], acc.at[idx_d[b]], ssem[b]).wait()

        istart(0, 0)
        istart(1, 1)
        iwait(0, 0)
        gstart(0)
        istart(2, 2)
        iwait(1, 1)
        gstart(1)

        pltpu.sync_copy(z_hbm, zbuf)
        r0 = s * RPT
        off = 0
        for n in [32] * (RPT // 32) + ([RPT % 32] if RPT % 32 else []):
            pltpu.sync_copy(zbuf.at[pl.ds(0, n)], acc.at[pl.ds(r0 + off, n)])
            off += n
        if TAIL:
            @pl.when(s == _NS - 1)
            def _zero_tail():
                pltpu.sync_copy(zbuf.at[pl.ds(0, TAIL)],
                                acc.at[pl.ds(_NS * RPT, TAIL)])
        plsc.subcore_barrier()

        gwait(0)
        sstart(0)
        istart(3, 3)
        iwait(2, 2)
        gstart(2)
        gwait(1)
        sstart(1)

        NMAIN = (NCH - 2 - 2) // 4

        def body(j, carry):
            i0 = 2 + 4 * j
            for t in range(4):
                b0 = (2 + t) % NBUF
                b1 = (3 + t) % NBUF
                b2 = t % NBUF
                swait(b2)
                istart(i0 + t + 2, b2)
                iwait(i0 + t + 1, b1)
                gstart(b1)
                gwait(b0)
                sstart(b0)
            return carry

        lax.fori_loop(0, NMAIN, body, 0)

        for i in range(2 + 4 * NMAIN, NCH):
            b0 = i % NBUF
            if i + 2 < NCH:
                b2 = (i + 2) % NBUF
                swait(b2)
                istart(i + 2, b2)
            if i + 1 < NCH:
                b1 = (i + 1) % NBUF
                iwait(i + 1, b1)
                gstart(b1)
            gwait(b0)
            sstart(b0)
        for b in range(NBUF):
            swait(b)
        plsc.subcore_barrier()

        pltpu.sync_copy(acc.at[pl.ds(r0, RPT)], out_hbm.at[c, pl.ds(r0, RPT)])
        if TAIL:
            @pl.when(s == _NS - 1)
            def _write_tail():
                pltpu.sync_copy(acc.at[pl.ds(_NS * RPT, TAIL)],
                                out_hbm.at[c, pl.ds(_NS * RPT, TAIL)])

    return hop


def _row_specs(R, D, n):
    return [pl.BlockSpec((R, D), lambda i: (i, 0)) for _ in range(n)]


def _w_spec(D):
    return pl.BlockSpec((D, D), lambda i: (0, 0))


def _b_spec(D):
    return pl.BlockSpec((1, D), lambda i: (0, 0))


def _f32(shape):
    return jax.ShapeDtypeStruct(shape, jnp.float32)


def _tc_begin(x, d0, d1, w0, R=1000):
    N, D = x.shape

    DW = d0.shape[1]

    def body(x_ref, d0_ref, d1_ref, w_ref, out_ref, p_ref, db_ref):
        deg = d0_ref[:, :1] + d1_ref[:, :1]
        dinv = jnp.where(deg > 0.0, lax.rsqrt(jnp.maximum(deg, 1.0)), 0.0)
        xv = x_ref[...]
        out_ref[...] = jnp.dot(xv, w_ref[...], preferred_element_type=jnp.float32)
        p_ref[...] = xv * dinv
        db_ref[...] = jnp.broadcast_to(dinv, xv.shape)

    dspec = pl.BlockSpec((R, DW), lambda i: (i, 0))
    return pl.pallas_call(
        body, grid=(N // R,),
        in_specs=[_row_specs(R, D, 1)[0], dspec, dspec, _w_spec(D)],
        out_specs=_row_specs(R, D, 3),
        out_shape=[_f32((N, D))] * 3,
    )(x, d0, d1, w0)


def _tc_p(s0, s1, db, R=1000):
    N, D = s0.shape

    def body(s0_ref, s1_ref, db_ref, p_ref):
        d = db_ref[...]
        p_ref[...] = (s0_ref[...] + s1_ref[...]) * d * d

    return pl.pallas_call(
        body, grid=(N // R,),
        in_specs=_row_specs(R, D, 3),
        out_specs=_row_specs(R, D, 1)[0],
        out_shape=_f32((N, D)),
    )(s0, s1, db)


def _tc_acc(s0, s1, db, w, oin, R=1000):
    N, D = s0.shape

    def body(s0_ref, s1_ref, db_ref, w_ref, oin_ref, oout_ref):
        h = (s0_ref[...] + s1_ref[...]) * db_ref[...]
        oout_ref[...] = oin_ref[...] + jnp.dot(
            h, w_ref[...], preferred_element_type=jnp.float32)

    return pl.pallas_call(
        body, grid=(N // R,),
        in_specs=_row_specs(R, D, 3) + [_w_spec(D)] + _row_specs(R, D, 1),
        out_specs=_row_specs(R, D, 1)[0],
        out_shape=_f32((N, D)),
    )(s0, s1, db, w, oin)


def _tc_layer_end(s0, s1, db, w, oin, b, wn, R=1000):
    N, D = s0.shape

    def body(s0_ref, s1_ref, db_ref, w_ref, oin_ref, b_ref, wn_ref,
             oout_ref, p_ref):
        d = db_ref[...]
        t = (s0_ref[...] + s1_ref[...]) * d
        h = jnp.tanh(oin_ref[...] + jnp.dot(
            t, w_ref[...], preferred_element_type=jnp.float32) + b_ref[...])
        oout_ref[...] = jnp.dot(h, wn_ref[...], preferred_element_type=jnp.float32)
        p_ref[...] = h * d

    return pl.pallas_call(
        body, grid=(N // R,),
        in_specs=(_row_specs(R, D, 3) + [_w_spec(D)] + _row_specs(R, D, 1)
                  + [_b_spec(D), _w_spec(D)]),
        out_specs=_row_specs(R, D, 2),
        out_shape=[_f32((N, D))] * 2,
    )(s0, s1, db, w, oin, b, wn)


def _tc_final(s0, s1, db, w, oin, b, wc, bc, R=1000):
    N, D = s0.shape

    def body(s0_ref, s1_ref, db_ref, w_ref, oin_ref, b_ref, wc_ref, bc_ref,
             out_ref):
        d = db_ref[...]
        t = (s0_ref[...] + s1_ref[...]) * d
        h = jnp.tanh(oin_ref[...] + jnp.dot(
            t, w_ref[...], preferred_element_type=jnp.float32) + b_ref[...])
        out_ref[...] = jnp.dot(
            h, wc_ref[...], preferred_element_type=jnp.float32) + bc_ref[...]

    return pl.pallas_call(
        body, grid=(N // R,),
        in_specs=(_row_specs(R, D, 3) + [_w_spec(D)] + _row_specs(R, D, 1)
                  + [_b_spec(D), _w_spec(D), _b_spec(D)]),
        out_specs=_row_specs(R, D, 1)[0],
        out_shape=_f32((N, D)),
    )(s0, s1, db, w, oin, b, wc, bc)


def kernel(x, edge_index, W1, b1, W2, b2, Wc, bc):
    N, D = x.shape
    E = edge_index.shape[1]
    src = edge_index[0]
    dst = edge_index[1]
    hop = _make_sc_hop(N, D, E)
    z = jnp.zeros((32, D), jnp.float32)

    dpart = hop(jnp.ones((N, D), jnp.float32), src, dst, z)

    b1r = b1.reshape(1, D)
    b2r = b2.reshape(1, D)
    bcr = bc.reshape(1, D)

    out, p, db = _tc_begin(x, dpart[0], dpart[1], W1[0])
    K1 = W1.shape[0] - 1
    K2 = W2.shape[0] - 1
    for k in range(1, K1 + 1):
        spart = hop(p, src, dst, z)
        if k < K1:
            p = _tc_p(spart[0], spart[1], db)
            out = _tc_acc(spart[0], spart[1], db, W1[k], out)
        else:
            out, p = _tc_layer_end(spart[0], spart[1], db, W1[k], out, b1r, W2[0])
    for k in range(1, K2 + 1):
        spart = hop(p, src, dst, z)
        if k < K2:
            p = _tc_p(spart[0], spart[1], db)
            out = _tc_acc(spart[0], spart[1], db, W2[k], out)
        else:
            return _tc_final(spart[0], spart[1], db, W2[k], out, b2r, Wc, bcr)


# speedup vs baseline: 1.0136x; 1.0136x over previous
import functools

import jax
import jax.numpy as jnp
from jax import lax
from jax.experimental import pallas as pl
from jax.experimental.pallas import tpu as pltpu
from jax.experimental.pallas import tpu_sc as plsc

_NC = 2
_NS = 16


@functools.lru_cache(maxsize=None)
def _make_sc_hop(N, D, E, CH=80):
    EPC = E // _NC
    EPT = EPC // _NS
    NCH = EPT // CH
    assert CH % 8 == 0 and NCH * CH == EPT and EPC * _NC == E
    RPT = (N // _NS) // 8 * 8
    TAIL = N - _NS * RPT
    assert TAIL % 8 == 0 and 0 <= TAIL <= 128

    mesh = plsc.VectorSubcoreMesh(
        core_axis_name="c", subcore_axis_name="s",
        num_cores=_NC, num_subcores=_NS)

    NBUF = 4
    assert NCH >= 6

    @functools.partial(
        pl.kernel,
        out_type=jax.ShapeDtypeStruct((_NC, N, D), jnp.float32),
        mesh=mesh,
        scratch_types=[
            [pltpu.VMEM((CH,), jnp.int32) for _ in range(NBUF)],
            [pltpu.VMEM((CH,), jnp.int32) for _ in range(NBUF)],
            [pltpu.VMEM((CH, D), jnp.float32) for _ in range(NBUF)],
            pltpu.VMEM((32, D), jnp.float32),
            pltpu.VMEM_SHARED((N, D), jnp.float32),
            [pltpu.SemaphoreType.DMA for _ in range(NBUF)],
            [pltpu.SemaphoreType.DMA for _ in range(NBUF)],
            [pltpu.SemaphoreType.DMA for _ in range(NBUF)],
        ],
    )
    def hop(p_hbm, src_hbm, dst_hbm, z_hbm, out_hbm,
            idx_s, idx_d, rows, zbuf, acc, gsem, ssem, isem):
        c = lax.axis_index("c")
        s = lax.axis_index("s")

        base = (c * _NS + s) * EPT

        def istart(i, b):
            e0 = base + i * CH
            pltpu.async_copy(src_hbm.at[pl.ds(e0, CH)], idx_s[b], isem[b])
            pltpu.async_copy(dst_hbm.at[pl.ds(e0, CH)], idx_d[b], isem[b])

        def iwait(i, b):
            e0 = base + i * CH
            pltpu.make_async_copy(
                src_hbm.at[pl.ds(e0, CH)], idx_s[b], isem[b]).wait()
            pltpu.make_async_copy(
                dst_hbm.at[pl.ds(e0, CH)], idx_d[b], isem[b]).wait()

        def gstart(b):
            pltpu.async_copy(p_hbm.at[idx_s[b]], rows[b], gsem[b])

        def gwait(b):
            pltpu.make_async_copy(p_hbm.at[idx_s[b]], rows[b], gsem[b]).wait()

        def sstart(b):
            pltpu.async_copy(rows[b], acc.at[idx_d[b]], ssem[b], add=True)

        def swait(b):
            pltpu.make_async_copy(rows[b], acc.at[idx_d[b]], ssem[b]).wait()

        istart(0, 0)
        istart(1, 1)
        iwait(0, 0)
        gstart(0)
        istart(2, 2)
        iwait(1, 1)
        gstart(1)

        pltpu.sync_copy(z_hbm, zbuf)
        r0 = s * RPT
        off = 0
        for n in [32] * (RPT // 32) + ([RPT % 32] if RPT % 32 else []):
            pltpu.sync_copy(zbuf.at[pl.ds(0, n)], acc.at[pl.ds(r0 + off, n)])
            off += n
        if TAIL:
            @pl.when(s == _NS - 1)
            def _zero_tail():
                pltpu.sync_copy(zbuf.at[pl.ds(0, TAIL)],
                                acc.at[pl.ds(_NS * RPT, TAIL)])
        plsc.subcore_barrier()

        gwait(0)
        sstart(0)
        istart(3, 3)
        iwait(2, 2)
        gstart(2)
        gwait(1)
        sstart(1)

        NMAIN = (NCH - 2 - 2) // 4

        def body(j, carry):
            i0 = 2 + 4 * j
            for t in range(4):
                b0 = (2 + t) % NBUF
                b1 = (3 + t) % NBUF
                b2 = t % NBUF
                swait(b2)
                istart(i0 + t + 2, b2)
                iwait(i0 + t + 1, b1)
                gstart(b1)
                gwait(b0)
                sstart(b0)
            return carry

        lax.fori_loop(0, NMAIN, body, 0)

        for i in range(2 + 4 * NMAIN, NCH):
            b0 = i % NBUF
            if i + 2 < NCH:
                b2 = (i + 2) % NBUF
                swait(b2)
                istart(i + 2, b2)
            if i + 1 < NCH:
                b1 = (i + 1) % NBUF
                iwait(i + 1, b1)
                gstart(b1)
            gwait(b0)
            sstart(b0)
        for b in range(NBUF):
            swait(b)
        plsc.subcore_barrier()

        pltpu.sync_copy(acc.at[pl.ds(r0, RPT)], out_hbm.at[c, pl.ds(r0, RPT)])
        if TAIL:
            @pl.when(s == _NS - 1)
            def _write_tail():
                pltpu.sync_copy(acc.at[pl.ds(_NS * RPT, TAIL)],
                                out_hbm.at[c, pl.ds(_NS * RPT, TAIL)])

    return hop


def _row_specs(R, D, n):
    return [pl.BlockSpec((R, D), lambda i: (i, 0)) for _ in range(n)]


def _w_spec(D):
    return pl.BlockSpec((D, D), lambda i: (0, 0))


def _b_spec(D):
    return pl.BlockSpec((1, D), lambda i: (0, 0))


def _f32(shape):
    return jax.ShapeDtypeStruct(shape, jnp.float32)


def _tc_begin(x, d0, d1, w0, R=1000):
    N, D = x.shape

    DW = d0.shape[1]

    def body(x_ref, d0_ref, d1_ref, w_ref, out_ref, p_ref, db_ref):
        deg = d0_ref[:, :1] + d1_ref[:, :1]
        dinv = jnp.where(deg > 0.0, lax.rsqrt(jnp.maximum(deg, 1.0)), 0.0)
        xv = x_ref[...]
        out_ref[...] = jnp.dot(xv, w_ref[...], preferred_element_type=jnp.float32)
        p_ref[...] = xv * dinv
        db_ref[...] = jnp.broadcast_to(dinv, xv.shape)

    dspec = pl.BlockSpec((R, DW), lambda i: (i, 0))
    return pl.pallas_call(
        body, grid=(N // R,),
        in_specs=[_row_specs(R, D, 1)[0], dspec, dspec, _w_spec(D)],
        out_specs=_row_specs(R, D, 3),
        out_shape=[_f32((N, D))] * 3,
    )(x, d0, d1, w0)


def _tc_mid(s0, s1, db, w, oin, R=1000):
    N, D = s0.shape

    def body(s0_ref, s1_ref, db_ref, w_ref, oin_ref, oout_ref, p_ref):
        d = db_ref[...]
        h = (s0_ref[...] + s1_ref[...]) * d
        oout_ref[...] = oin_ref[...] + jnp.dot(
            h, w_ref[...], preferred_element_type=jnp.float32)
        p_ref[...] = h * d

    return pl.pallas_call(
        body, grid=(N // R,),
        in_specs=_row_specs(R, D, 3) + [_w_spec(D)] + _row_specs(R, D, 1),
        out_specs=_row_specs(R, D, 2),
        out_shape=[_f32((N, D))] * 2,
    )(s0, s1, db, w, oin)


def _tc_layer_end(s0, s1, db, w, oin, b, wn, R=1000):
    N, D = s0.shape

    def body(s0_ref, s1_ref, db_ref, w_ref, oin_ref, b_ref, wn_ref,
             oout_ref, p_ref):
        d = db_ref[...]
        t = (s0_ref[...] + s1_ref[...]) * d
        h = jnp.tanh(oin_ref[...] + jnp.dot(
            t, w_ref[...], preferred_element_type=jnp.float32) + b_ref[...])
        oout_ref[...] = jnp.dot(h, wn_ref[...], preferred_element_type=jnp.float32)
        p_ref[...] = h * d

    return pl.pallas_call(
        body, grid=(N // R,),
        in_specs=(_row_specs(R, D, 3) + [_w_spec(D)] + _row_specs(R, D, 1)
                  + [_b_spec(D), _w_spec(D)]),
        out_specs=_row_specs(R, D, 2),
        out_shape=[_f32((N, D))] * 2,
    )(s0, s1, db, w, oin, b, wn)


def _tc_final(s0, s1, db, w, oin, b, wc, bc, R=1000):
    N, D = s0.shape

    def body(s0_ref, s1_ref, db_ref, w_ref, oin_ref, b_ref, wc_ref, bc_ref,
             out_ref):
        d = db_ref[...]
        t = (s0_ref[...] + s1_ref[...]) * d
        h = jnp.tanh(oin_ref[...] + jnp.dot(
            t, w_ref[...], preferred_element_type=jnp.float32) + b_ref[...])
        out_ref[...] = jnp.dot(
            h, wc_ref[...], preferred_element_type=jnp.float32) + bc_ref[...]

    return pl.pallas_call(
        body, grid=(N // R,),
        in_specs=(_row_specs(R, D, 3) + [_w_spec(D)] + _row_specs(R, D, 1)
                  + [_b_spec(D), _w_spec(D), _b_spec(D)]),
        out_specs=_row_specs(R, D, 1)[0],
        out_shape=_f32((N, D)),
    )(s0, s1, db, w, oin, b, wc, bc)


def kernel(x, edge_index, W1, b1, W2, b2, Wc, bc):
    N, D = x.shape
    E = edge_index.shape[1]
    src = edge_index[0]
    dst = edge_index[1]
    hop = _make_sc_hop(N, D, E)
    z = jnp.zeros((32, D), jnp.float32)

    dpart = hop(jnp.ones((N, D), jnp.float32), src, dst, z)

    b1r = b1.reshape(1, D)
    b2r = b2.reshape(1, D)
    bcr = bc.reshape(1, D)

    out, p, db = _tc_begin(x, dpart[0], dpart[1], W1[0])
    K1 = W1.shape[0] - 1
    K2 = W2.shape[0] - 1
    for k in range(1, K1 + 1):
        spart = hop(p, src, dst, z)
        if k < K1:
            out, p = _tc_mid(spart[0], spart[1], db, W1[k], out)
        else:
            out, p = _tc_layer_end(spart[0], spart[1], db, W1[k], out, b1r, W2[0])
    for k in range(1, K2 + 1):
        spart = hop(p, src, dst, z)
        if k < K2:
            out, p = _tc_mid(spart[0], spart[1], db, W2[k], out)
        else:
            return _tc_final(spart[0], spart[1], db, W2[k], out, b2r, Wc, bcr)
